# baseline (speedup 1.0000x reference)
import functools

import jax
import jax.numpy as jnp
from jax import lax
from jax.experimental import pallas as pl
from jax.experimental.pallas import tpu as pltpu
from jax.experimental.pallas import tpu_sc as plsc

_SC_CHUNKS_PER_WORKER = 0
_CH = 8192
_NW = 32
_LANES = 16

_LOG2_POLY = (
    -3.028317481039271,
    6.065830143185771,
    -5.2641104770847,
    3.2188328370634505,
    -1.2342631730389073,
    0.2668588228611466,
    -0.024825606614389147,
)
_LN2 = 0.6931471805599453


def _focal_terms_tanh(x, t):
    th = jnp.tanh(0.5 * x)
    p = jnp.clip(0.5 + 0.5 * th, 1e-4, 1.0 - 1e-4)
    pos = t >= 1.0
    arg = jnp.where(pos, p, 1.0 - p)
    omp = 1.0 - p
    omt = 1.0 - t
    omt2 = omt * omt
    w = jnp.where(pos, omp * omp, (p * p) * (omt2 * omt2))
    return arg, w, pos


def _focal_terms(x, t):
    p = jnp.clip(1.0 / (1.0 + jnp.exp(-x)), 1e-4, 1.0 - 1e-4)
    pos = t >= 1.0
    arg = jnp.where(pos, p, 1.0 - p)
    omp = 1.0 - p
    omt = 1.0 - t
    omt2 = omt * omt
    w = jnp.where(pos, omp * omp, (p * p) * (omt2 * omt2))
    return arg, w, pos


def _tc_body(x_ref, t_ref, out_ref):
    i = pl.program_id(0)
    arg, w, pos = _focal_terms_tanh(x_ref[...], t_ref[...])
    part = jnp.sum(jnp.log(arg) * w)
    cnt = jnp.sum(jnp.where(pos, 1.0, 0.0))

    @pl.when(i == 0)
    def _init():
        out_ref[0] = part
        out_ref[1] = cnt

    @pl.when(i > 0)
    def _acc():
        out_ref[0] += part
        out_ref[1] += cnt


def _tc_sums(x2, t2, grid):
    rows, w = x2.shape
    blk = rows // grid
    return pl.pallas_call(
        _tc_body,
        grid=(grid,),
        in_specs=[
            pl.BlockSpec((blk, w), lambda i: (i, 0)),
            pl.BlockSpec((blk, w), lambda i: (i, 0)),
        ],
        out_specs=pl.BlockSpec(memory_space=pltpu.SMEM),
        out_shape=jax.ShapeDtypeStruct((2,), jnp.float32),
    )(x2, t2)


def _sc_log(arg):
    bits = lax.bitcast_convert_type(arg, jnp.int32)
    e = lax.convert_element_type(
        lax.shift_right_arithmetic(bits, 23) - 127, jnp.float32
    )
    m = lax.bitcast_convert_type(
        lax.bitwise_or(lax.bitwise_and(bits, 0x007FFFFF), 0x3F800000),
        jnp.float32,
    )
    acc = jnp.full((_LANES,), _LOG2_POLY[6], jnp.float32)
    for k in range(5, -1, -1):
        acc = acc * m + jnp.float32(_LOG2_POLY[k])
    return (e + acc) * jnp.float32(_LN2)


def _sc_body(nchunks, x_hbm, t_hbm, out_hbm, xbuf, tbuf, stage, sem):
    wid = lax.axis_index("s") * 2 + lax.axis_index("c")
    base = wid * (nchunks * _CH)

    def chunk_step(c, carry):
        acc, cnt = carry
        off = base + c * _CH
        pltpu.sync_copy(x_hbm.at[pl.ds(off, _CH)], xbuf)
        pltpu.sync_copy(t_hbm.at[pl.ds(off, _CH)], tbuf)

        def vec_step(i, carry2):
            acc2, cnt2 = carry2
            xv = xbuf[pl.ds(i * _LANES, _LANES)]
            tv = tbuf[pl.ds(i * _LANES, _LANES)]
            arg, w, pos = _focal_terms(xv, tv)
            acc2 = acc2 + _sc_log(arg) * w
            cnt2 = cnt2 + jnp.where(pos, 1.0, 0.0)
            return acc2, cnt2

        return lax.fori_loop(0, _CH // _LANES, vec_step, (acc, cnt))

    zero = jnp.zeros((_LANES,), jnp.float32)
    acc, cnt = lax.fori_loop(0, nchunks, chunk_step, (zero, zero))
    stage[0, :] = acc
    stage[1, :] = cnt
    pltpu.sync_copy(stage, out_hbm.at[wid])


def _sc_sums(x_flat, t_flat, nchunks):
    mesh = plsc.VectorSubcoreMesh(core_axis_name="c", subcore_axis_name="s")
    kern = functools.partial(
        pl.kernel,
        mesh=mesh,
        out_type=jax.ShapeDtypeStruct((_NW, 2, _LANES), jnp.float32),
        scratch_types=[
            pltpu.VMEM((_CH,), jnp.float32),
            pltpu.VMEM((_CH,), jnp.float32),
            pltpu.VMEM((2, _LANES), jnp.float32),
            pltpu.SemaphoreType.DMA,
        ],
    )(functools.partial(_sc_body, nchunks))
    return kern(x_flat, t_flat)


def kernel(main_kf_logits, heatmap_target):
    shape = main_kf_logits.shape
    n = shape[0] * shape[1] * shape[2] * shape[3]
    w = shape[3]
    n_sc = _SC_CHUNKS_PER_WORKER * _CH * _NW
    loss_sum = jnp.float32(0.0)
    cnt_sum = jnp.float32(0.0)
    if n_sc:
        x_flat = main_kf_logits.reshape(n)
        t_flat = heatmap_target.reshape(n)
        sc = _sc_sums(x_flat[:n_sc], t_flat[:n_sc], _SC_CHUNKS_PER_WORKER)
        loss_sum = loss_sum + jnp.sum(sc[:, 0, :])
        cnt_sum = cnt_sum + jnp.sum(sc[:, 1, :])
    if n_sc < n:
        tc_w = 512
        rows = (n - n_sc) // tc_w
        x2 = main_kf_logits.reshape(n)[n_sc:].reshape(rows, tc_w)
        t2 = heatmap_target.reshape(n)[n_sc:].reshape(rows, tc_w)
        grid = 48
        while rows % grid:
            grid -= 1
        tc = _tc_sums(x2, t2, grid)
        loss_sum = loss_sum + tc[0]
        cnt_sum = cnt_sum + tc[1]
    return -loss_sum / jnp.maximum(cnt_sum, 1.0)


# speedup vs baseline: 1.4046x; 1.4046x over previous
import functools

import jax
import jax.numpy as jnp
from jax import lax
from jax.experimental import pallas as pl
from jax.experimental.pallas import tpu as pltpu
from jax.experimental.pallas import tpu_sc as plsc

_SC_CHUNKS_PER_WORKER = 0
_CH = 8192
_NW = 32
_LANES = 16

_LOG2_POLY = (
    -3.028317481039271,
    6.065830143185771,
    -5.2641104770847,
    3.2188328370634505,
    -1.2342631730389073,
    0.2668588228611466,
    -0.024825606614389147,
)
_LN2 = 0.6931471805599453


def _focal_terms_tanh(x, t):
    th = jnp.tanh(0.5 * x)
    p = jnp.clip(0.5 + 0.5 * th, 1e-4, 1.0 - 1e-4)
    pos = t >= 1.0
    arg = jnp.where(pos, p, 1.0 - p)
    omp = 1.0 - p
    omt = 1.0 - t
    omt2 = omt * omt
    w = jnp.where(pos, omp * omp, (p * p) * (omt2 * omt2))
    return arg, w, pos


def _focal_terms(x, t):
    p = jnp.clip(1.0 / (1.0 + jnp.exp(-x)), 1e-4, 1.0 - 1e-4)
    pos = t >= 1.0
    arg = jnp.where(pos, p, 1.0 - p)
    omp = 1.0 - p
    omt = 1.0 - t
    omt2 = omt * omt
    w = jnp.where(pos, omp * omp, (p * p) * (omt2 * omt2))
    return arg, w, pos


def _tc_body(x_ref, t_ref, out_ref):
    i = pl.program_id(0)
    arg, w, pos = _focal_terms_tanh(x_ref[...], t_ref[...])
    contrib = jnp.log(arg) * w
    cntv = jnp.where(pos, 1.0, 0.0)
    blk, lanes = contrib.shape
    part = jnp.sum(contrib.reshape(blk // 8, 8, lanes), axis=0)
    cnt = jnp.sum(cntv.reshape(blk // 8, 8, lanes), axis=0)

    @pl.when(i == 0)
    def _init():
        out_ref[0] = part
        out_ref[1] = cnt

    @pl.when(i > 0)
    def _acc():
        out_ref[0] += part
        out_ref[1] += cnt


def _tc_sums(x2, t2, grid):
    rows, w = x2.shape
    blk = rows // grid
    out = pl.pallas_call(
        _tc_body,
        grid=(grid,),
        in_specs=[
            pl.BlockSpec((blk, w), lambda i: (i, 0)),
            pl.BlockSpec((blk, w), lambda i: (i, 0)),
        ],
        out_specs=pl.BlockSpec((2, 8, w), lambda i: (0, 0, 0)),
        out_shape=jax.ShapeDtypeStruct((2, 8, w), jnp.float32),
    )(x2, t2)
    return jnp.sum(out[0]), jnp.sum(out[1])


def _sc_log(arg):
    bits = lax.bitcast_convert_type(arg, jnp.int32)
    e = lax.convert_element_type(
        lax.shift_right_arithmetic(bits, 23) - 127, jnp.float32
    )
    m = lax.bitcast_convert_type(
        lax.bitwise_or(lax.bitwise_and(bits, 0x007FFFFF), 0x3F800000),
        jnp.float32,
    )
    acc = jnp.full((_LANES,), _LOG2_POLY[6], jnp.float32)
    for k in range(5, -1, -1):
        acc = acc * m + jnp.float32(_LOG2_POLY[k])
    return (e + acc) * jnp.float32(_LN2)


def _sc_body(nchunks, x_hbm, t_hbm, out_hbm, xbuf, tbuf, stage, sem):
    wid = lax.axis_index("s") * 2 + lax.axis_index("c")
    base = wid * (nchunks * _CH)

    def chunk_step(c, carry):
        acc, cnt = carry
        off = base + c * _CH
        pltpu.sync_copy(x_hbm.at[pl.ds(off, _CH)], xbuf)
        pltpu.sync_copy(t_hbm.at[pl.ds(off, _CH)], tbuf)

        def vec_step(i, carry2):
            acc2, cnt2 = carry2
            xv = xbuf[pl.ds(i * _LANES, _LANES)]
            tv = tbuf[pl.ds(i * _LANES, _LANES)]
            arg, w, pos = _focal_terms(xv, tv)
            acc2 = acc2 + _sc_log(arg) * w
            cnt2 = cnt2 + jnp.where(pos, 1.0, 0.0)
            return acc2, cnt2

        return lax.fori_loop(0, _CH // _LANES, vec_step, (acc, cnt))

    zero = jnp.zeros((_LANES,), jnp.float32)
    acc, cnt = lax.fori_loop(0, nchunks, chunk_step, (zero, zero))
    stage[0, :] = acc
    stage[1, :] = cnt
    pltpu.sync_copy(stage, out_hbm.at[wid])


def _sc_sums(x_flat, t_flat, nchunks):
    mesh = plsc.VectorSubcoreMesh(core_axis_name="c", subcore_axis_name="s")
    kern = functools.partial(
        pl.kernel,
        mesh=mesh,
        out_type=jax.ShapeDtypeStruct((_NW, 2, _LANES), jnp.float32),
        scratch_types=[
            pltpu.VMEM((_CH,), jnp.float32),
            pltpu.VMEM((_CH,), jnp.float32),
            pltpu.VMEM((2, _LANES), jnp.float32),
            pltpu.SemaphoreType.DMA,
        ],
    )(functools.partial(_sc_body, nchunks))
    return kern(x_flat, t_flat)


def kernel(main_kf_logits, heatmap_target):
    shape = main_kf_logits.shape
    n = shape[0] * shape[1] * shape[2] * shape[3]
    w = shape[3]
    n_sc = _SC_CHUNKS_PER_WORKER * _CH * _NW
    loss_sum = jnp.float32(0.0)
    cnt_sum = jnp.float32(0.0)
    if n_sc:
        x_flat = main_kf_logits.reshape(n)
        t_flat = heatmap_target.reshape(n)
        sc = _sc_sums(x_flat[:n_sc], t_flat[:n_sc], _SC_CHUNKS_PER_WORKER)
        loss_sum = loss_sum + jnp.sum(sc[:, 0, :])
        cnt_sum = cnt_sum + jnp.sum(sc[:, 1, :])
    if n_sc < n:
        tc_w = 512
        rows = (n - n_sc) // tc_w
        x2 = main_kf_logits.reshape(n)[n_sc:].reshape(rows, tc_w)
        t2 = heatmap_target.reshape(n)[n_sc:].reshape(rows, tc_w)
        grid = 12
        while rows % grid:
            grid -= 1
        tc = _tc_sums(x2, t2, grid)
        loss_sum = loss_sum + tc[0]
        cnt_sum = cnt_sum + tc[1]
    return -loss_sum / jnp.maximum(cnt_sum, 1.0)


# speedup vs baseline: 3.1113x; 2.2151x over previous
import functools

import jax
import jax.numpy as jnp
from jax import lax
from jax.experimental import pallas as pl
from jax.experimental.pallas import tpu as pltpu
from jax.experimental.pallas import tpu_sc as plsc

_SC_CHUNKS_PER_WORKER = 0
_CH = 8192
_NW = 32
_LANES = 16

_LOG2_POLY = (
    -3.028317481039271,
    6.065830143185771,
    -5.2641104770847,
    3.2188328370634505,
    -1.2342631730389073,
    0.2668588228611466,
    -0.024825606614389147,
)
_LN2 = 0.6931471805599453


def _focal_terms_tanh(x, t):
    th = jnp.tanh(0.5 * x)
    p = jnp.clip(0.5 + 0.5 * th, 1e-4, 1.0 - 1e-4)
    pos = t >= 1.0
    arg = jnp.where(pos, p, 1.0 - p)
    omp = 1.0 - p
    omt = 1.0 - t
    omt2 = omt * omt
    w = jnp.where(pos, omp * omp, (p * p) * (omt2 * omt2))
    return arg, w, pos


def _focal_terms(x, t):
    p = jnp.clip(1.0 / (1.0 + jnp.exp(-x)), 1e-4, 1.0 - 1e-4)
    pos = t >= 1.0
    arg = jnp.where(pos, p, 1.0 - p)
    omp = 1.0 - p
    omt = 1.0 - t
    omt2 = omt * omt
    w = jnp.where(pos, omp * omp, (p * p) * (omt2 * omt2))
    return arg, w, pos


def _tc_body(x_ref, t_ref, out_ref):
    i = pl.program_id(0)
    x = x_ref[...]
    bb, c, h, lanes = x.shape
    rows = bb * c * h
    arg, w, pos = _focal_terms_tanh(x.reshape(rows, lanes), t_ref[...].reshape(rows, lanes))
    contrib = jnp.log(arg) * w
    cntv = jnp.where(pos, 1.0, 0.0)
    part = jnp.sum(contrib.reshape(rows // 8, 8, lanes), axis=0)
    cnt = jnp.sum(cntv.reshape(rows // 8, 8, lanes), axis=0)

    @pl.when(i == 0)
    def _init():
        out_ref[0] = part
        out_ref[1] = cnt

    @pl.when(i > 0)
    def _acc():
        out_ref[0] += part
        out_ref[1] += cnt


def _tc_sums(x4, t4, grid):
    b, c, h, w = x4.shape
    blk = b // grid
    out = pl.pallas_call(
        _tc_body,
        grid=(grid,),
        in_specs=[
            pl.BlockSpec((blk, c, h, w), lambda i: (i, 0, 0, 0)),
            pl.BlockSpec((blk, c, h, w), lambda i: (i, 0, 0, 0)),
        ],
        out_specs=pl.BlockSpec((2, 8, w), lambda i: (0, 0, 0)),
        out_shape=jax.ShapeDtypeStruct((2, 8, w), jnp.float32),
    )(x4, t4)
    return jnp.sum(out[0]), jnp.sum(out[1])


def _sc_log(arg):
    bits = lax.bitcast_convert_type(arg, jnp.int32)
    e = lax.convert_element_type(
        lax.shift_right_arithmetic(bits, 23) - 127, jnp.float32
    )
    m = lax.bitcast_convert_type(
        lax.bitwise_or(lax.bitwise_and(bits, 0x007FFFFF), 0x3F800000),
        jnp.float32,
    )
    acc = jnp.full((_LANES,), _LOG2_POLY[6], jnp.float32)
    for k in range(5, -1, -1):
        acc = acc * m + jnp.float32(_LOG2_POLY[k])
    return (e + acc) * jnp.float32(_LN2)


def _sc_body(nchunks, x_hbm, t_hbm, out_hbm, xbuf, tbuf, stage, sem):
    wid = lax.axis_index("s") * 2 + lax.axis_index("c")
    base = wid * (nchunks * _CH)

    def chunk_step(c, carry):
        acc, cnt = carry
        off = base + c * _CH
        pltpu.sync_copy(x_hbm.at[pl.ds(off, _CH)], xbuf)
        pltpu.sync_copy(t_hbm.at[pl.ds(off, _CH)], tbuf)

        def vec_step(i, carry2):
            acc2, cnt2 = carry2
            xv = xbuf[pl.ds(i * _LANES, _LANES)]
            tv = tbuf[pl.ds(i * _LANES, _LANES)]
            arg, w, pos = _focal_terms(xv, tv)
            acc2 = acc2 + _sc_log(arg) * w
            cnt2 = cnt2 + jnp.where(pos, 1.0, 0.0)
            return acc2, cnt2

        return lax.fori_loop(0, _CH // _LANES, vec_step, (acc, cnt))

    zero = jnp.zeros((_LANES,), jnp.float32)
    acc, cnt = lax.fori_loop(0, nchunks, chunk_step, (zero, zero))
    stage[0, :] = acc
    stage[1, :] = cnt
    pltpu.sync_copy(stage, out_hbm.at[wid])


def _sc_sums(x_flat, t_flat, nchunks):
    mesh = plsc.VectorSubcoreMesh(core_axis_name="c", subcore_axis_name="s")
    kern = functools.partial(
        pl.kernel,
        mesh=mesh,
        out_type=jax.ShapeDtypeStruct((_NW, 2, _LANES), jnp.float32),
        scratch_types=[
            pltpu.VMEM((_CH,), jnp.float32),
            pltpu.VMEM((_CH,), jnp.float32),
            pltpu.VMEM((2, _LANES), jnp.float32),
            pltpu.SemaphoreType.DMA,
        ],
    )(functools.partial(_sc_body, nchunks))
    return kern(x_flat, t_flat)


def kernel(main_kf_logits, heatmap_target):
    shape = main_kf_logits.shape
    n = shape[0] * shape[1] * shape[2] * shape[3]
    w = shape[3]
    n_sc = _SC_CHUNKS_PER_WORKER * _CH * _NW
    loss_sum = jnp.float32(0.0)
    cnt_sum = jnp.float32(0.0)
    if n_sc:
        x_flat = main_kf_logits.reshape(n)
        t_flat = heatmap_target.reshape(n)
        sc = _sc_sums(x_flat[:n_sc], t_flat[:n_sc], _SC_CHUNKS_PER_WORKER)
        loss_sum = loss_sum + jnp.sum(sc[:, 0, :])
        cnt_sum = cnt_sum + jnp.sum(sc[:, 1, :])
    if n_sc < n:
        assert n_sc == 0, "hybrid split handled along batch dim"
        grid = 8
        tc = _tc_sums(main_kf_logits, heatmap_target, grid)
        loss_sum = loss_sum + tc[0]
        cnt_sum = cnt_sum + tc[1]
    return -loss_sum / jnp.maximum(cnt_sum, 1.0)


# speedup vs baseline: 3.1701x; 1.0189x over previous
import functools

import jax
import jax.numpy as jnp
from jax import lax
from jax.experimental import pallas as pl
from jax.experimental.pallas import tpu as pltpu
from jax.experimental.pallas import tpu_sc as plsc

_SC_CHUNKS_PER_WORKER = 0
_CH = 8192
_NW = 32
_LANES = 16

_LOG2_POLY = (
    -3.028317481039271,
    6.065830143185771,
    -5.2641104770847,
    3.2188328370634505,
    -1.2342631730389073,
    0.2668588228611466,
    -0.024825606614389147,
)
_LN2 = 0.6931471805599453


def _focal_terms_tanh(x, t):
    pos = t >= 1.0
    th = jnp.tanh(0.5 * x)
    s = jnp.where(pos, th, -th)
    arg = jnp.clip(0.5 + 0.5 * s, 1e-4, 1.0 - 1e-4)
    oma = 1.0 - arg
    omt = 1.0 - t
    omt2 = omt * omt
    w = (oma * oma) * jnp.where(pos, 1.0, omt2 * omt2)
    return arg, w, pos


def _focal_terms(x, t):
    p = jnp.clip(1.0 / (1.0 + jnp.exp(-x)), 1e-4, 1.0 - 1e-4)
    pos = t >= 1.0
    arg = jnp.where(pos, p, 1.0 - p)
    omp = 1.0 - p
    omt = 1.0 - t
    omt2 = omt * omt
    w = jnp.where(pos, omp * omp, (p * p) * (omt2 * omt2))
    return arg, w, pos


def _tc_body(x_ref, t_ref, out_ref):
    i = pl.program_id(0)
    x = x_ref[...]
    bb, c, h, lanes = x.shape
    rows = bb * c * h
    arg, w, pos = _focal_terms_tanh(x.reshape(rows, lanes), t_ref[...].reshape(rows, lanes))
    contrib = jnp.log2(arg) * w
    cntv = jnp.where(pos, 1.0, 0.0)
    part = jnp.sum(contrib.reshape(rows // 8, 8, lanes), axis=0)
    cnt = jnp.sum(cntv.reshape(rows // 8, 8, lanes), axis=0)

    @pl.when(i == 0)
    def _init():
        out_ref[0] = part
        out_ref[1] = cnt

    @pl.when(i > 0)
    def _acc():
        out_ref[0] += part
        out_ref[1] += cnt


def _tc_sums(x4, t4, grid):
    b, c, h, w = x4.shape
    blk = b // grid
    out = pl.pallas_call(
        _tc_body,
        grid=(grid,),
        in_specs=[
            pl.BlockSpec((blk, c, h, w), lambda i: (i, 0, 0, 0)),
            pl.BlockSpec((blk, c, h, w), lambda i: (i, 0, 0, 0)),
        ],
        out_specs=pl.BlockSpec((2, 8, w), lambda i: (0, 0, 0)),
        out_shape=jax.ShapeDtypeStruct((2, 8, w), jnp.float32),
    )(x4, t4)
    return jnp.sum(out[0]), jnp.sum(out[1])


def _sc_log(arg):
    bits = lax.bitcast_convert_type(arg, jnp.int32)
    e = lax.convert_element_type(
        lax.shift_right_arithmetic(bits, 23) - 127, jnp.float32
    )
    m = lax.bitcast_convert_type(
        lax.bitwise_or(lax.bitwise_and(bits, 0x007FFFFF), 0x3F800000),
        jnp.float32,
    )
    acc = jnp.full((_LANES,), _LOG2_POLY[6], jnp.float32)
    for k in range(5, -1, -1):
        acc = acc * m + jnp.float32(_LOG2_POLY[k])
    return (e + acc) * jnp.float32(_LN2)


def _sc_body(nchunks, x_hbm, t_hbm, out_hbm, xbuf, tbuf, stage, sem):
    wid = lax.axis_index("s") * 2 + lax.axis_index("c")
    base = wid * (nchunks * _CH)

    def chunk_step(c, carry):
        acc, cnt = carry
        off = base + c * _CH
        pltpu.sync_copy(x_hbm.at[pl.ds(off, _CH)], xbuf)
        pltpu.sync_copy(t_hbm.at[pl.ds(off, _CH)], tbuf)

        def vec_step(i, carry2):
            acc2, cnt2 = carry2
            xv = xbuf[pl.ds(i * _LANES, _LANES)]
            tv = tbuf[pl.ds(i * _LANES, _LANES)]
            arg, w, pos = _focal_terms(xv, tv)
            acc2 = acc2 + _sc_log(arg) * w
            cnt2 = cnt2 + jnp.where(pos, 1.0, 0.0)
            return acc2, cnt2

        return lax.fori_loop(0, _CH // _LANES, vec_step, (acc, cnt))

    zero = jnp.zeros((_LANES,), jnp.float32)
    acc, cnt = lax.fori_loop(0, nchunks, chunk_step, (zero, zero))
    stage[0, :] = acc
    stage[1, :] = cnt
    pltpu.sync_copy(stage, out_hbm.at[wid])


def _sc_sums(x_flat, t_flat, nchunks):
    mesh = plsc.VectorSubcoreMesh(core_axis_name="c", subcore_axis_name="s")
    kern = functools.partial(
        pl.kernel,
        mesh=mesh,
        out_type=jax.ShapeDtypeStruct((_NW, 2, _LANES), jnp.float32),
        scratch_types=[
            pltpu.VMEM((_CH,), jnp.float32),
            pltpu.VMEM((_CH,), jnp.float32),
            pltpu.VMEM((2, _LANES), jnp.float32),
            pltpu.SemaphoreType.DMA,
        ],
    )(functools.partial(_sc_body, nchunks))
    return kern(x_flat, t_flat)


def kernel(main_kf_logits, heatmap_target):
    shape = main_kf_logits.shape
    n = shape[0] * shape[1] * shape[2] * shape[3]
    w = shape[3]
    n_sc = _SC_CHUNKS_PER_WORKER * _CH * _NW
    loss_sum = jnp.float32(0.0)
    cnt_sum = jnp.float32(0.0)
    if n_sc:
        x_flat = main_kf_logits.reshape(n)
        t_flat = heatmap_target.reshape(n)
        sc = _sc_sums(x_flat[:n_sc], t_flat[:n_sc], _SC_CHUNKS_PER_WORKER)
        loss_sum = loss_sum + jnp.sum(sc[:, 0, :])
        cnt_sum = cnt_sum + jnp.sum(sc[:, 1, :])
    if n_sc < n:
        assert n_sc == 0, "hybrid split handled along batch dim"
        grid = 8
        tc = _tc_sums(main_kf_logits, heatmap_target, grid)
        loss_sum = loss_sum + tc[0] * jnp.float32(_LN2)
        cnt_sum = cnt_sum + tc[1]
    return -loss_sum / jnp.maximum(cnt_sum, 1.0)
